# baseline (speedup 1.0000x reference)
import jax
import jax.numpy as jnp
from jax.experimental import pallas as pl
from jax.experimental.pallas import tpu as pltpu


def _make_dma_body(B, S, D, SBLK, NBUF):
    NB = S // SBLK

    def body(w_hbm, o_hbm, buf, in_sem, out_sem):
        def in_copy(i):
            slot = i % NBUF
            return pltpu.make_async_copy(
                w_hbm.at[pl.ds(i * SBLK, SBLK), :], buf.at[slot], in_sem.at[slot]
            )

        def out_copy(i, b):
            slot = i % NBUF
            return pltpu.make_async_copy(
                buf.at[slot],
                o_hbm.at[b, pl.ds(i * SBLK, SBLK), :],
                out_sem.at[slot, b],
            )

        in_copy(0).start()
        for i in range(NB):
            in_copy(i).wait()
            if i + 1 < NB:
                if i + 1 - NBUF >= 0:
                    for b in range(B):
                        out_copy(i + 1 - NBUF, b).wait()
                in_copy(i + 1).start()
            for b in range(B):
                out_copy(i, b).start()
        for i in range(max(0, NB - NBUF), NB):
            for b in range(B):
                out_copy(i, b).wait()

    return body


def kernel(x, W):
    B, S = x.shape
    D = W.shape[1]
    SBLK = 1024
    NBUF = 4
    assert S % SBLK == 0 and S // SBLK >= NBUF
    out = pl.pallas_call(
        _make_dma_body(B, S, D, SBLK, NBUF),
        in_specs=[pl.BlockSpec(memory_space=pl.ANY)],
        out_specs=pl.BlockSpec(memory_space=pl.ANY),
        out_shape=jax.ShapeDtypeStruct((B, S, D), W.dtype),
        scratch_shapes=[
            pltpu.VMEM((NBUF, SBLK, D), W.dtype),
            pltpu.SemaphoreType.DMA((NBUF,)),
            pltpu.SemaphoreType.DMA((NBUF, B)),
        ],
    )(W[:S])
    return out
